# baseline (speedup 1.0000x reference)
import itertools
import math
from functools import partial

import numpy as np
import jax
import jax.numpy as jnp
from jax.experimental import pallas as pl
from jax.experimental.pallas import tpu as pltpu

_G = 12
_NUM_GRAPHS = 16


def _a4_tables():
    perms = []
    for p in itertools.permutations(range(4)):
        inv = sum(1 for a in range(4) for b in range(a + 1, 4) if p[a] > p[b])
        if inv % 2 == 0:
            perms.append(p)
    idx = {p: i for i, p in enumerate(perms)}

    def compose(p, q):
        return tuple(p[q[i]] for i in range(4))

    def pinv(p):
        out = [0] * 4
        for i, pi in enumerate(p):
            out[pi] = i
        return tuple(out)

    cay = np.array(
        [[idx[compose(pinv(perms[h]), perms[g])] for h in range(_G)] for g in range(_G)],
        dtype=np.int32,
    )
    V = np.array(
        [[1.0, 1.0, 1.0], [1.0, -1.0, -1.0], [-1.0, 1.0, -1.0], [-1.0, -1.0, 1.0]],
        dtype=np.float64,
    ) / np.sqrt(3.0)
    rots = np.stack(
        [np.linalg.lstsq(V, V[list(p)], rcond=None)[0].T for p in perms]
    ).astype(np.float32)
    return cay, rots


_CAY, _ROT = _a4_tables()


def _rope_mats(GH, D):
    P = D // 2
    TH, E = GH * P, GH * D
    sw = np.zeros((E, E), np.float32)
    dup = np.zeros((TH, E), np.float32)
    for gh in range(GH):
        for p in range(P):
            t = gh * P + p
            a = gh * D + 2 * p
            dup[t, a] = -1.0
            dup[t, a + 1] = 1.0
            sw[a + 1, a] = 1.0
            sw[a, a + 1] = 1.0
    r = np.arange(E)
    bd = (r[:, None] // D == r[None, :] // D).astype(np.float32)
    return sw, bd, dup


def _dense_w(W):
    wf = W[jnp.asarray(_CAY)]
    return jnp.transpose(wf, (1, 3, 0, 2)).reshape(_G * W.shape[2], _G * W.shape[1])


def _phase_a(x_ref, pos_ref, b_ref, wq_ref, wv_ref, bq_ref, bv_ref, f_ref,
             sw_ref, bd_ref, qout_ref, m_ref, *, nblocks):
    i = pl.program_id(0)

    @pl.when(i == 0)
    def _init():
        m_ref[...] = jnp.zeros_like(m_ref)

    xb = x_ref[...]
    q = jnp.dot(xb, wq_ref[...], preferred_element_type=jnp.float32) + bq_ref[0:1, :]
    v = jnp.dot(xb, wv_ref[...], preferred_element_type=jnp.float32) + bv_ref[0:1, :]
    th = jnp.dot(pos_ref[...], f_ref[...], preferred_element_type=jnp.float32)
    cosE = jnp.cos(th)
    sinE = jnp.sin(th)
    qsw = jnp.dot(q, sw_ref[...], preferred_element_type=jnp.float32)
    q_rot = q * cosE + qsw * sinE
    k_rot = cosE + sinE
    qout_ref[...] = q_rot

    bvals = b_ref[...]
    bmin = jnp.min(bvals)
    bmax = jnp.max(bvals)
    for sid in range(_NUM_GRAPHS):
        @pl.when(jnp.logical_and(sid >= bmin, sid <= bmax))
        def _acc(sid=sid):
            msk = (bvals == sid).astype(jnp.float32)
            vm = v * msk
            contrib = jax.lax.dot_general(
                k_rot, vm, (((0,), (0,)), ((), ())),
                preferred_element_type=jnp.float32)
            m_ref[sid] = m_ref[sid] + contrib

    @pl.when(i == nblocks - 1)
    def _mask_bd():
        m_ref[...] = m_ref[...] * bd_ref[...][None]


def _phase_b(q_ref, b_ref, m_ref, wo_ref, bo_ref, out_ref, acc_ref):
    bvals = b_ref[...]
    bmin = jnp.min(bvals)
    bmax = jnp.max(bvals)
    acc_ref[...] = jnp.zeros_like(acc_ref)
    qb = q_ref[...]
    for sid in range(_NUM_GRAPHS):
        @pl.when(jnp.logical_and(sid >= bmin, sid <= bmax))
        def _acc(sid=sid):
            msk = (bvals == sid).astype(jnp.float32)
            acc_ref[...] += jnp.dot(qb * msk, m_ref[sid],
                                    preferred_element_type=jnp.float32)
    out_ref[...] = (jnp.dot(acc_ref[...], wo_ref[...],
                            preferred_element_type=jnp.float32) + bo_ref[0:1, :])


def kernel(x, pos, batch, Wq, bq, Wv, bv, freqs, Wo, bo):
    N, IN = x.shape
    H, P, _ = freqs.shape
    D = 2 * P
    GH = _G * H
    E = GH * D
    TH = GH * P
    OUT = _G * Wo.shape[1]

    B = 2000
    nblocks = -(-N // B)
    Np = nblocks * B

    wq_d = _dense_w(Wq)
    wv_d = _dense_w(Wv)
    wo_d = _dense_w(Wo)
    bq8 = jnp.tile(jnp.tile(bq, _G)[None, :], (8, 1))
    bv8 = jnp.tile(jnp.tile(bv, _G)[None, :], (8, 1))
    bo8 = jnp.tile(jnp.tile(bo, _G)[None, :], (8, 1))
    sw, bd, dup = _rope_mats(GH, D)
    f3 = jnp.einsum('gij,hpi->ghpj', jnp.asarray(_ROT), freqs).reshape(TH, 3).T
    f8 = jnp.zeros((8, E), jnp.float32).at[:3, :].set(f3 @ jnp.asarray(dup))

    x_p = x if Np == N else jnp.pad(x, ((0, Np - N), (0, 0)))
    pos8 = jnp.zeros((Np, 8), jnp.float32).at[:N, :3].set(pos)
    b2d = jnp.pad(batch.astype(jnp.int32), (0, Np - N),
                  constant_values=_NUM_GRAPHS).reshape(Np, 1)

    full = lambda shape: pl.BlockSpec(shape, lambda i: (0,) * len(shape))

    q_rot, M = pl.pallas_call(
        partial(_phase_a, nblocks=nblocks),
        grid=(nblocks,),
        in_specs=[
            pl.BlockSpec((B, IN), lambda i: (i, 0)),
            pl.BlockSpec((B, 8), lambda i: (i, 0)),
            pl.BlockSpec((B, 1), lambda i: (i, 0)),
            full((IN, E)), full((IN, E)),
            full((8, E)), full((8, E)),
            full((8, E)),
            full((E, E)), full((E, E)),
        ],
        out_specs=[
            pl.BlockSpec((B, E), lambda i: (i, 0)),
            pl.BlockSpec((_NUM_GRAPHS, E, E), lambda i: (0, 0, 0)),
        ],
        out_shape=[
            jax.ShapeDtypeStruct((Np, E), jnp.float32),
            jax.ShapeDtypeStruct((_NUM_GRAPHS, E, E), jnp.float32),
        ],
    )(x_p, pos8, b2d, wq_d, wv_d, bq8, bv8, f8, sw, bd)

    out = pl.pallas_call(
        _phase_b,
        grid=(nblocks,),
        in_specs=[
            pl.BlockSpec((B, E), lambda i: (i, 0)),
            pl.BlockSpec((B, 1), lambda i: (i, 0)),
            full((_NUM_GRAPHS, E, E)),
            full((E, OUT)),
            full((8, OUT)),
        ],
        out_specs=pl.BlockSpec((B, OUT), lambda i: (i, 0)),
        out_shape=jax.ShapeDtypeStruct((Np, OUT), jnp.float32),
        scratch_shapes=[pltpu.VMEM((B, E), jnp.float32)],
    )(q_rot, b2d, M, wo_d, bo8)

    return out[:N]


# speedup vs baseline: 1.2308x; 1.2308x over previous
import itertools
import math
from functools import partial

import numpy as np
import jax
import jax.numpy as jnp
from jax.experimental import pallas as pl
from jax.experimental.pallas import tpu as pltpu

_G = 12
_NUM_GRAPHS = 16


def _a4_tables():
    perms = []
    for p in itertools.permutations(range(4)):
        inv = sum(1 for a in range(4) for b in range(a + 1, 4) if p[a] > p[b])
        if inv % 2 == 0:
            perms.append(p)
    idx = {p: i for i, p in enumerate(perms)}

    def compose(p, q):
        return tuple(p[q[i]] for i in range(4))

    def pinv(p):
        out = [0] * 4
        for i, pi in enumerate(p):
            out[pi] = i
        return tuple(out)

    cay = np.array(
        [[idx[compose(pinv(perms[h]), perms[g])] for h in range(_G)] for g in range(_G)],
        dtype=np.int32,
    )
    V = np.array(
        [[1.0, 1.0, 1.0], [1.0, -1.0, -1.0], [-1.0, 1.0, -1.0], [-1.0, -1.0, 1.0]],
        dtype=np.float64,
    ) / np.sqrt(3.0)
    rots = np.stack(
        [np.linalg.lstsq(V, V[list(p)], rcond=None)[0].T for p in perms]
    ).astype(np.float32)
    return cay, rots


_CAY, _ROT = _a4_tables()


def _rope_mats(GH, D):
    P = D // 2
    TH, E = GH * P, GH * D
    ec = np.zeros((TH, E), np.float32)
    es = np.zeros((TH, E), np.float32)
    sw = np.zeros((E, E), np.float32)
    for gh in range(GH):
        for p in range(P):
            t = gh * P + p
            a = gh * D + 2 * p
            ec[t, a] = 1.0
            ec[t, a + 1] = 1.0
            es[t, a] = -1.0
            es[t, a + 1] = 1.0
            sw[a + 1, a] = 1.0
            sw[a, a + 1] = 1.0
    r = np.arange(E)
    bd = (r[:, None] // D == r[None, :] // D).astype(np.float32)
    return ec, es, sw, bd


def _dense_w(W):
    wf = W[jnp.asarray(_CAY)]
    return jnp.transpose(wf, (1, 3, 0, 2)).reshape(_G * W.shape[2], _G * W.shape[1])


def _phase_a(x_ref, pos_ref, b_ref, wq_ref, wv_ref, bq_ref, bv_ref, f_ref,
             ec_ref, es_ref, sw_ref, bd_ref, wo_ref, qout_ref, mo_ref, m_ref,
             *, nblocks):
    i = pl.program_id(0)

    @pl.when(i == 0)
    def _init():
        m_ref[...] = jnp.zeros_like(m_ref)

    bf = jnp.bfloat16
    xb = x_ref[...].astype(bf)
    q = jnp.dot(xb, wq_ref[...], preferred_element_type=jnp.float32) + bq_ref[0:1, :]
    v = jnp.dot(xb, wv_ref[...], preferred_element_type=jnp.float32)
    th = jnp.dot(pos_ref[...], f_ref[...], preferred_element_type=jnp.float32)
    cosE = jnp.dot(jnp.cos(th).astype(bf), ec_ref[...], preferred_element_type=jnp.float32)
    sinE = jnp.dot(jnp.sin(th).astype(bf), es_ref[...], preferred_element_type=jnp.float32)
    qsw = jnp.dot(q.astype(bf), sw_ref[...], preferred_element_type=jnp.float32)
    q_rot = q * cosE + qsw * sinE
    k_rot = (cosE + sinE).astype(bf)
    v = (v + bv_ref[0:1, :]).astype(bf)
    qout_ref[...] = q_rot.astype(bf)

    bvals = b_ref[...]
    bmin = jnp.min(bvals)
    bmax = jnp.max(bvals)

    @pl.when(bmin == bmax)
    def _fast():
        contrib = jax.lax.dot_general(
            k_rot, v, (((0,), (0,)), ((), ())),
            preferred_element_type=jnp.float32)
        m_ref[pl.ds(bmin, 1)] = m_ref[pl.ds(bmin, 1)] + contrib[None]

    @pl.when(bmin < bmax)
    def _slow():
        for sid in range(_NUM_GRAPHS):
            @pl.when(jnp.logical_and(sid >= bmin, sid <= bmax))
            def _acc(sid=sid):
                msk = (bvals == sid).astype(jnp.float32)
                contrib = jax.lax.dot_general(
                    k_rot, v * msk, (((0,), (0,)), ((), ())),
                    preferred_element_type=jnp.float32)
                m_ref[sid] = m_ref[sid] + contrib

    @pl.when(i == nblocks - 1)
    def _finalize():
        for sid in range(_NUM_GRAPHS):
            mo_ref[sid] = jnp.dot(
                (m_ref[sid] * bd_ref[...]).astype(jnp.bfloat16), wo_ref[...],
                preferred_element_type=jnp.float32).astype(jnp.bfloat16)


def _phase_b(q_ref, b_ref, mo_ref, bo_ref, out_ref):
    bf = jnp.bfloat16
    bvals = b_ref[...]
    bmin = jnp.min(bvals)
    bmax = jnp.max(bvals)
    qb = q_ref[...]

    @pl.when(bmin == bmax)
    def _fast():
        mb = mo_ref[pl.ds(bmin, 1)].reshape(mo_ref.shape[1], mo_ref.shape[2])
        out_ref[...] = (jnp.dot(qb, mb, preferred_element_type=jnp.float32)
                        + bo_ref[0:1, :])

    @pl.when(bmin < bmax)
    def _slow():
        out_ref[...] = jnp.broadcast_to(bo_ref[0:1, :], out_ref.shape)
        for sid in range(_NUM_GRAPHS):
            @pl.when(jnp.logical_and(sid >= bmin, sid <= bmax))
            def _acc(sid=sid):
                msk = (bvals == sid).astype(bf)
                out_ref[...] += jnp.dot(qb * msk, mo_ref[sid],
                                        preferred_element_type=jnp.float32)


def kernel(x, pos, batch, Wq, bq, Wv, bv, freqs, Wo, bo):
    N, IN = x.shape
    H, P, _ = freqs.shape
    D = 2 * P
    GH = _G * H
    E = GH * D
    TH = GH * P
    OUT = _G * Wo.shape[1]

    B = 1000
    nblocks = -(-N // B)
    Np = nblocks * B

    bf = jnp.bfloat16
    wq_d = _dense_w(Wq).astype(bf)
    wv_d = _dense_w(Wv).astype(bf)
    wo_d = _dense_w(Wo).astype(bf)
    bq8 = jnp.tile(jnp.tile(bq, _G)[None, :], (8, 1))
    bv8 = jnp.tile(jnp.tile(bv, _G)[None, :], (8, 1))
    bo8 = jnp.tile(jnp.tile(bo, _G)[None, :], (8, 1))
    ec, es, sw, bd = _rope_mats(GH, D)
    ec16 = jnp.asarray(ec, dtype=bf)
    es16 = jnp.asarray(es, dtype=bf)
    sw16 = jnp.asarray(sw, dtype=bf)
    f3 = jnp.einsum('gij,hpi->ghpj', jnp.asarray(_ROT), freqs).reshape(TH, 3).T
    f8 = jnp.zeros((8, TH), jnp.float32).at[:3, :].set(f3)

    x_p = x if Np == N else jnp.pad(x, ((0, Np - N), (0, 0)))
    pos8 = jnp.zeros((Np, 8), jnp.float32).at[:N, :3].set(pos)
    b2d = jnp.pad(batch.astype(jnp.int32), (0, Np - N),
                  constant_values=_NUM_GRAPHS).reshape(Np, 1)

    full = lambda shape: pl.BlockSpec(shape, lambda i: (0,) * len(shape))

    q_rot, MO = pl.pallas_call(
        partial(_phase_a, nblocks=nblocks),
        grid=(nblocks,),
        in_specs=[
            pl.BlockSpec((B, IN), lambda i: (i, 0)),
            pl.BlockSpec((B, 8), lambda i: (i, 0)),
            pl.BlockSpec((B, 1), lambda i: (i, 0)),
            full((IN, E)), full((IN, E)),
            full((8, E)), full((8, E)),
            full((8, TH)),
            full((TH, E)), full((TH, E)),
            full((E, E)), full((E, E)),
            full((E, OUT)),
        ],
        out_specs=[
            pl.BlockSpec((B, E), lambda i: (i, 0)),
            pl.BlockSpec((_NUM_GRAPHS, E, OUT), lambda i: (0, 0, 0)),
        ],
        out_shape=[
            jax.ShapeDtypeStruct((Np, E), bf),
            jax.ShapeDtypeStruct((_NUM_GRAPHS, E, OUT), bf),
        ],
        scratch_shapes=[pltpu.VMEM((_NUM_GRAPHS, E, E), jnp.float32)],
    )(x_p, pos8, b2d, wq_d, wv_d, bq8, bv8, f8, ec16, es16, sw16, bd, wo_d)

    out = pl.pallas_call(
        _phase_b,
        grid=(nblocks,),
        in_specs=[
            pl.BlockSpec((B, E), lambda i: (i, 0)),
            pl.BlockSpec((B, 1), lambda i: (i, 0)),
            full((_NUM_GRAPHS, E, OUT)),
            full((8, OUT)),
        ],
        out_specs=pl.BlockSpec((B, OUT), lambda i: (i, 0)),
        out_shape=jax.ShapeDtypeStruct((Np, OUT), jnp.float32),
    )(q_rot, b2d, MO, bo8)

    return out[:N]


# speedup vs baseline: 1.7824x; 1.4481x over previous
import itertools
import math
from functools import partial

import numpy as np
import jax
import jax.numpy as jnp
from jax.experimental import pallas as pl
from jax.experimental.pallas import tpu as pltpu

_G = 12
_NUM_GRAPHS = 16


def _a4_tables():
    perms = []
    for p in itertools.permutations(range(4)):
        inv = sum(1 for a in range(4) for b in range(a + 1, 4) if p[a] > p[b])
        if inv % 2 == 0:
            perms.append(p)
    idx = {p: i for i, p in enumerate(perms)}

    def compose(p, q):
        return tuple(p[q[i]] for i in range(4))

    def pinv(p):
        out = [0] * 4
        for i, pi in enumerate(p):
            out[pi] = i
        return tuple(out)

    cay = np.array(
        [[idx[compose(pinv(perms[h]), perms[g])] for h in range(_G)] for g in range(_G)],
        dtype=np.int32,
    )
    V = np.array(
        [[1.0, 1.0, 1.0], [1.0, -1.0, -1.0], [-1.0, 1.0, -1.0], [-1.0, -1.0, 1.0]],
        dtype=np.float64,
    ) / np.sqrt(3.0)
    rots = np.stack(
        [np.linalg.lstsq(V, V[list(p)], rcond=None)[0].T for p in perms]
    ).astype(np.float32)
    return cay, rots


_CAY, _ROT = _a4_tables()

_COS6 = (0.9999994437075935, -0.49999558228580177, 0.04166103351910408,
         -0.0013862749961056388, 2.4253229890178196e-05, -2.2194129828401188e-07)
_SIN5 = (0.9999845934509939, -0.1666325937682276, 0.008312388279695326,
         -0.00019316269888625115, 2.1732569600863005e-06)


def _fast_sincos(th):
    t = th * jnp.float32(1.0 / (2.0 * math.pi))
    t = t - jnp.round(t)
    r = t * jnp.float32(2.0 * math.pi)
    r2 = r * r
    c = jnp.float32(_COS6[5])
    for k in (4, 3, 2, 1, 0):
        c = c * r2 + jnp.float32(_COS6[k])
    s = jnp.float32(_SIN5[4])
    for k in (3, 2, 1, 0):
        s = s * r2 + jnp.float32(_SIN5[k])
    return s * r, c


def _rope_mats(GH, D):
    P = D // 2
    TH, E = GH * P, GH * D
    ec = np.zeros((TH, E), np.float32)
    es = np.zeros((TH, E), np.float32)
    sw = np.zeros((E, E), np.float32)
    for gh in range(GH):
        for p in range(P):
            t = gh * P + p
            a = gh * D + 2 * p
            ec[t, a] = 1.0
            ec[t, a + 1] = 1.0
            es[t, a] = -1.0
            es[t, a + 1] = 1.0
            sw[a + 1, a] = 1.0
            sw[a, a + 1] = 1.0
    r = np.arange(E)
    bd = (r[:, None] // D == r[None, :] // D).astype(np.float32)
    return ec, es, sw, bd


def _dense_w(W):
    wf = W[jnp.asarray(_CAY)]
    return jnp.transpose(wf, (1, 3, 0, 2)).reshape(_G * W.shape[2], _G * W.shape[1])


def _fused(x_ref, pos_ref, b_ref, wq_ref, wv_ref, bq_ref, bv_ref, f_ref,
           ec_ref, es_ref, sw_ref, bd_ref, wo_ref, bo_ref, out_ref,
           q_sc, mo_sc, m_sc, *, nblocks, B):
    i = pl.program_id(0)
    bf = jnp.bfloat16

    @pl.when(i == 0)
    def _init():
        m_sc[...] = jnp.zeros_like(m_sc)

    bvals = b_ref[...]
    bmin = jnp.min(bvals)
    bmax = jnp.max(bvals)

    @pl.when(i < nblocks)
    def _build():
        xb = x_ref[...].astype(bf)
        q = jnp.dot(xb, wq_ref[...], preferred_element_type=jnp.float32) + bq_ref[0:1, :]
        v = jnp.dot(xb, wv_ref[...], preferred_element_type=jnp.float32)
        pb = pos_ref[...]
        th = (pb[:, 0:1] * f_ref[0:1, :] + pb[:, 1:2] * f_ref[1:2, :]
              + pb[:, 2:3] * f_ref[2:3, :])
        sinv, cosv = _fast_sincos(th)
        cosE = jnp.dot(cosv.astype(bf), ec_ref[...], preferred_element_type=jnp.float32)
        sinE = jnp.dot(sinv.astype(bf), es_ref[...], preferred_element_type=jnp.float32)
        qsw = jnp.dot(q.astype(bf), sw_ref[...], preferred_element_type=jnp.float32)
        q_rot = q * cosE + qsw * sinE
        k_rot = (cosE + sinE).astype(bf)
        vb = (v + bv_ref[0:1, :]).astype(bf)
        q_sc[pl.ds(i * B, B), :] = q_rot.astype(bf)

        @pl.when(bmin == bmax)
        def _fast():
            contrib = jax.lax.dot_general(
                k_rot, vb, (((0,), (0,)), ((), ())),
                preferred_element_type=jnp.float32)
            m_sc[pl.ds(bmin, 1)] = m_sc[pl.ds(bmin, 1)] + contrib[None]

        @pl.when(bmin < bmax)
        def _slow():
            def _acc(sid, carry):
                msk = (bvals == sid).astype(bf)
                contrib = jax.lax.dot_general(
                    k_rot, vb * msk, (((0,), (0,)), ((), ())),
                    preferred_element_type=jnp.float32)
                m_sc[pl.ds(sid, 1)] = m_sc[pl.ds(sid, 1)] + contrib[None]
                return carry
            jax.lax.fori_loop(bmin, bmax + 1, _acc, 0)

        @pl.when(i == nblocks - 1)
        def _finalize():
            for sid in range(_NUM_GRAPHS):
                mo_sc[sid] = jnp.dot(
                    (m_sc[sid] * bd_ref[...]).astype(bf), wo_ref[...],
                    preferred_element_type=jnp.float32).astype(bf)

    @pl.when(i >= nblocks)
    def _readout():
        j = i - nblocks
        qb = q_sc[pl.ds(j * B, B), :]

        @pl.when(bmin == bmax)
        def _fast():
            mb = mo_sc[pl.ds(bmin, 1)].reshape(mo_sc.shape[1], mo_sc.shape[2])
            out_ref[...] = (jnp.dot(qb, mb, preferred_element_type=jnp.float32)
                            + bo_ref[0:1, :])

        @pl.when(bmin < bmax)
        def _slow():
            out_ref[...] = jnp.broadcast_to(bo_ref[0:1, :], out_ref.shape)

            def _acc(sid, carry):
                msk = (bvals == sid).astype(bf)
                mb = mo_sc[pl.ds(sid, 1)].reshape(mo_sc.shape[1], mo_sc.shape[2])
                out_ref[...] += jnp.dot(qb * msk, mb,
                                        preferred_element_type=jnp.float32)
                return carry
            jax.lax.fori_loop(bmin, bmax + 1, _acc, 0)


def kernel(x, pos, batch, Wq, bq, Wv, bv, freqs, Wo, bo):
    N, IN = x.shape
    H, P, _ = freqs.shape
    D = 2 * P
    GH = _G * H
    E = GH * D
    TH = GH * P
    OUT = _G * Wo.shape[1]

    B = 2000
    nblocks = -(-N // B)
    Np = nblocks * B

    bf = jnp.bfloat16
    wq_d = _dense_w(Wq).astype(bf)
    wv_d = _dense_w(Wv).astype(bf)
    wo_d = _dense_w(Wo).astype(bf)
    bq8 = jnp.tile(jnp.tile(bq, _G)[None, :], (8, 1))
    bv8 = jnp.tile(jnp.tile(bv, _G)[None, :], (8, 1))
    bo8 = jnp.tile(jnp.tile(bo, _G)[None, :], (8, 1))
    ec, es, sw, bd = _rope_mats(GH, D)
    ec16 = jnp.asarray(ec, dtype=bf)
    es16 = jnp.asarray(es, dtype=bf)
    sw16 = jnp.asarray(sw, dtype=bf)
    f3 = jnp.einsum('gij,hpi->ghpj', jnp.asarray(_ROT), freqs).reshape(TH, 3).T
    f8 = jnp.zeros((8, TH), jnp.float32).at[:3, :].set(f3)

    x_p = x if Np == N else jnp.pad(x, ((0, Np - N), (0, 0)))
    pos_p = pos if Np == N else jnp.pad(pos, ((0, Np - N), (0, 0)))
    b2d = jnp.pad(batch.astype(jnp.int32), (0, Np - N),
                  constant_values=_NUM_GRAPHS).reshape(Np, 1)

    full = lambda shape: pl.BlockSpec(shape, lambda i: (0,) * len(shape))
    nb = nblocks
    blk = lambda i: (jnp.where(i < nb, i, i - nb), 0)
    blk_a = lambda i: (jnp.minimum(i, nb - 1), 0)

    out = pl.pallas_call(
        partial(_fused, nblocks=nb, B=B),
        grid=(2 * nb,),
        in_specs=[
            pl.BlockSpec((B, IN), blk_a),
            pl.BlockSpec((B, 3), blk_a),
            pl.BlockSpec((B, 1), blk),
            full((IN, E)), full((IN, E)),
            full((8, E)), full((8, E)),
            full((8, TH)),
            full((TH, E)), full((TH, E)),
            full((E, E)), full((E, E)),
            full((E, OUT)), full((8, OUT)),
        ],
        out_specs=pl.BlockSpec((B, OUT), lambda i: (jnp.where(i < nb, 0, i - nb), 0)),
        out_shape=jax.ShapeDtypeStruct((Np, OUT), jnp.float32),
        scratch_shapes=[
            pltpu.VMEM((Np, E), bf),
            pltpu.VMEM((_NUM_GRAPHS, E, OUT), bf),
            pltpu.VMEM((_NUM_GRAPHS, E, E), jnp.float32),
        ],
    )(x_p, pos_p, b2d, wq_d, wv_d, bq8, bv8, f8, ec16, es16, sw16, bd,
      wo_d, bo8)

    return out[:N]
